# baseline (speedup 1.0000x reference)
import functools

import jax
import jax.numpy as jnp
from jax import lax
from jax.experimental import pallas as pl
from jax.experimental.pallas import tpu as pltpu
from jax.experimental.pallas import tpu_sc as plsc

KT = 512
TT = 256


def _bf16_dot(w_bf, r_part):
    return lax.dot_general(
        w_bf, r_part.astype(jnp.bfloat16), (((1,), (1,)), ((), ())),
        preferred_element_type=jnp.float32)


def _argmin_body(w_ref, r_ref, rsq_ref, idx_ref, best_d, best_i, *, n_k):
    k = pl.program_id(1)

    @pl.when(k == 0)
    def _init():
        best_d[...] = jnp.full((1, TT), jnp.inf, jnp.float32)
        best_i[...] = jnp.zeros((1, TT), jnp.int32)

    w = w_ref[...]
    wsq = jnp.sum(w * w, axis=1)
    w_bf = w.astype(jnp.bfloat16)
    kio = lax.broadcasted_iota(jnp.int32, (KT, TT), 0) + k * KT
    r = r_ref[...]
    cross = _bf16_dot(w_bf, r)
    r2 = rsq_ref[0]
    d2 = (r2 - 2.0 * cross) + wsq[:, None]
    m = jnp.min(d2, axis=0)[None, :]
    ii = jnp.min(jnp.where(d2 == m, kio, jnp.int32(2**30)),
                 axis=0)[None, :]
    bd = best_d[...]
    upd = m < bd
    best_d[...] = jnp.where(upd, m, bd)
    best_i[...] = jnp.where(upd, ii, best_i[...])

    @pl.when(k == n_k - 1)
    def _flush():
        idx_ref[...] = best_i[...].reshape(1, 1, TT)


def _tc_argmin(res, rsq, w):
    N, D = res.shape
    K = w.shape[0]
    n_k = K // KT
    n_t = N // TT
    return pl.pallas_call(
        functools.partial(_argmin_body, n_k=n_k),
        grid=(n_t, n_k),
        in_specs=[
            pl.BlockSpec((KT, D), lambda t, k: (k, 0)),
            pl.BlockSpec((TT, D), lambda t, k: (t, 0)),
            pl.BlockSpec((1, 1, TT), lambda t, k: (t, 0, 0)),
        ],
        out_specs=pl.BlockSpec((1, 1, TT), lambda t, k: (t, 0, 0)),
        out_shape=jax.ShapeDtypeStruct((n_t, 1, TT), jnp.int32),
        scratch_shapes=[pltpu.VMEM((1, TT), jnp.float32),
                        pltpu.VMEM((1, TT), jnp.int32)],
    )(w, res, rsq.reshape(n_t, 1, TT))


def _sc_gather_sub(table, idx, res):
    N, D = res.shape
    info = plsc.get_sparse_core_info()
    NC, NS = info.num_cores, info.num_subcores
    bpw = N // (NC * NS)
    mesh = plsc.VectorSubcoreMesh(core_axis_name="c", subcore_axis_name="s")

    def body(table_hbm, idx_hbm, res_hbm, out_hbm, idx_v, rows_v, res_v, sem):
        wid = lax.axis_index("s") * NC + lax.axis_index("c")
        base = wid * bpw
        pltpu.sync_copy(idx_hbm.at[pl.ds(base, bpw)], idx_v)
        pltpu.async_copy(table_hbm.at[idx_v], rows_v, sem).wait()
        pltpu.sync_copy(res_hbm.at[pl.ds(base, bpw)], res_v)

        def row(i, carry):
            for j in range(D // 16):
                s = pl.ds(j * 16, 16)
                res_v[i, s] = res_v[i, s] - rows_v[i, s]
            return carry

        lax.fori_loop(0, bpw, row, 0)
        pltpu.sync_copy(res_v, out_hbm.at[pl.ds(base, bpw)])

    f = pl.kernel(
        body,
        out_type=jax.ShapeDtypeStruct((N, D), jnp.float32),
        mesh=mesh,
        scratch_types=[pltpu.VMEM((bpw,), jnp.int32),
                       pltpu.VMEM((bpw, D), jnp.float32),
                       pltpu.VMEM((bpw, D), jnp.float32),
                       pltpu.SemaphoreType.DMA],
    )
    return f(table, idx, res)


def _sc_gather_sub_final(table, idx, res, x):
    N, D = res.shape
    info = plsc.get_sparse_core_info()
    NC, NS = info.num_cores, info.num_subcores
    bpw = N // (NC * NS)
    mesh = plsc.VectorSubcoreMesh(core_axis_name="c", subcore_axis_name="s")

    def body(table_hbm, idx_hbm, res_hbm, x_hbm, out_hbm,
             idx_v, rows_v, res_v, x_v, sem):
        wid = lax.axis_index("s") * NC + lax.axis_index("c")
        base = wid * bpw
        pltpu.sync_copy(idx_hbm.at[pl.ds(base, bpw)], idx_v)
        pltpu.async_copy(table_hbm.at[idx_v], rows_v, sem).wait()
        pltpu.sync_copy(res_hbm.at[pl.ds(base, bpw)], res_v)
        pltpu.sync_copy(x_hbm.at[pl.ds(base, bpw)], x_v)

        def row(i, carry):
            for j in range(D // 16):
                s = pl.ds(j * 16, 16)
                x_v[i, s] = x_v[i, s] - res_v[i, s] + rows_v[i, s]
            return carry

        lax.fori_loop(0, bpw, row, 0)
        pltpu.sync_copy(x_v, out_hbm.at[pl.ds(base, bpw)])

    f = pl.kernel(
        body,
        out_type=jax.ShapeDtypeStruct((N, D), jnp.float32),
        mesh=mesh,
        scratch_types=[pltpu.VMEM((bpw,), jnp.int32),
                       pltpu.VMEM((bpw, D), jnp.float32),
                       pltpu.VMEM((bpw, D), jnp.float32),
                       pltpu.VMEM((bpw, D), jnp.float32),
                       pltpu.SemaphoreType.DMA],
    )
    return f(table, idx, res, x)


def kernel(input, codebooks):
    B, D, T = input.shape
    L, K, _ = codebooks.shape
    x = jnp.transpose(input, (0, 2, 1)).reshape(B * T, D)
    res = x
    idx_list = []
    out_flat = None
    for l in range(L):
        rsq = jnp.sum(res * res, axis=-1)
        idx3 = _tc_argmin(res, rsq, codebooks[l])
        idx_flat = idx3.reshape(-1)
        idx_list.append(idx3.reshape(B, T))
        if l + 1 < L:
            res = _sc_gather_sub(codebooks[l], idx_flat, res)
        else:
            out_flat = _sc_gather_sub_final(codebooks[l], idx_flat, res, x)
    out = out_flat.reshape(B, T, D).transpose(0, 2, 1).reshape(input.shape)
    indices = jnp.stack(idx_list, axis=1).reshape((B, L, T))
    return out, indices
